# Initial kernel scaffold; baseline (speedup 1.0000x reference)
import functools

import jax
import jax.numpy as jnp
from jax import lax
from jax.experimental import pallas as pl
from jax.experimental.pallas import tpu as pltpu

N = 4096
ROWS, COLS = 32, 128
THRESHOLD = 0.5
D_O = 0.74 * 1.4
D_H = 0.528 * 1.4
EXPAND = (3.0 / 4.0, 25.0 / 32.0, 25.0 / 32.0)
BIG_I = jnp.int32(1 << 24)
SENT = jnp.int32(1 << 24)


def _feasible_offsets():
    offs = []
    for dz in range(-2, 3):
        for dx in range(-2, 3):
            for dy in range(-2, 3):
                if dz == dx == dy == 0:
                    continue
                mind2 = ((max(abs(dz) - 1, 0) * EXPAND[0]) ** 2
                         + (max(abs(dx) - 1, 0) * EXPAND[1]) ** 2
                         + (max(abs(dy) - 1, 0) * EXPAND[2]) ** 2)
                if mind2 < D_O * D_O:
                    offs.append(dz * 1024 + dx * 32 + dy)
    return offs


NBR_OFFS = _feasible_offsets()
N_NMS = len(NBR_OFFS)
MATCH_OFFS = [0] + NBR_OFFS
N_MATCH = len(MATCH_OFFS)


def _roll_flat(a, d_pos):
    q = d_pos // COLS
    rem = d_pos % COLS
    b1 = pltpu.roll(a, (ROWS - q % ROWS) % ROWS, axis=0)
    b2 = pltpu.roll(b1, ROWS - 1, axis=0)
    c1 = pltpu.roll(b1, (COLS - rem) % COLS, axis=1)
    c2 = pltpu.roll(b2, (COLS - rem) % COLS, axis=1)
    col = lax.broadcasted_iota(jnp.int32, a.shape, 1)
    return jnp.where(col < COLS - rem, c1, c2)


def _body(pred_ref, targ_ref, offs_ref, out_ref, mpp_ref, mtt_ref, mpt_ref):
    s = pl.program_id(0)
    e = s % 2
    thr = jnp.where(e == 0, jnp.float32(D_O), jnp.float32(D_H))

    flat = (lax.broadcasted_iota(jnp.int32, (ROWS, COLS), 0) * COLS
            + lax.broadcasted_iota(jnp.int32, (ROWS, COLS), 1))
    zc = (flat // 1024).astype(jnp.float32)
    xc = ((flat // 32) % 32).astype(jnp.float32)
    yc = (flat % 32).astype(jnp.float32)

    def coords(ref):
        z = (ref[0, 0] + zc) * jnp.float32(EXPAND[0])
        x = (ref[0, 1] + xc) * jnp.float32(EXPAND[1])
        y = (ref[0, 2] + yc) * jnp.float32(EXPAND[2])
        return z, x, y, ref[0, 3]

    pz, px, py, ps = coords(pred_ref)
    tz, tx, ty, ts = coords(targ_ref)

    def close(az, ax, ay, bz, bx, by, d_pos):
        gz = _roll_flat(bz, d_pos)
        gx = _roll_flat(bx, d_pos)
        gy = _roll_flat(by, d_pos)
        dz = gz - az
        dx = gx - ax
        dy = gy - ay
        d2 = dz * dz + dx * dx + dy * dy
        return jnp.sqrt(d2) < thr

    def precompute(t, _):
        d = offs_ref[0, t]
        d_pos = jnp.where(d < 0, d + N, d)
        neg = d < 0
        cpp = close(pz, px, py, pz, px, py, d_pos)
        ctt = close(tz, tx, ty, tz, tx, ty, d_pos)
        cpt = close(pz, px, py, tz, tx, ty, d_pos)
        gps = _roll_flat(ps, d_pos)
        gts = _roll_flat(ts, d_pos)
        ep = (gps > ps) | ((gps == ps) & neg)
        et = (gts > ts) | ((gts == ts) & neg)
        mpt_ref[t] = cpt.astype(jnp.float32)

        @pl.when(t > 0)
        def _():
            mpp_ref[t - 1] = (cpp & ep).astype(jnp.float32)
            mtt_ref[t - 1] = (ctt & et).astype(jnp.float32)

        return 0

    lax.fori_loop(0, N_MATCH, precompute, 0)

    def nms(valid, mask_ref):
        def body(carry):
            sel, _ = carry

            def acc(t, sup):
                d = offs_ref[0, t + 1]
                d_pos = jnp.where(d < 0, d + N, d)
                return jnp.maximum(sup, mask_ref[t] * _roll_flat(sel, d_pos))

            sup = lax.fori_loop(0, N_NMS, acc, jnp.zeros_like(sel))
            new = valid * (1.0 - sup)
            changed = jnp.sum(jnp.abs(new - sel)) > 0.0
            return new, changed

        sel, _ = lax.while_loop(lambda c: c[1], body,
                                (valid, jnp.bool_(True)))
        return sel

    valid_p = (ps > THRESHOLD).astype(jnp.float32)
    valid_t = (ts > THRESHOLD).astype(jnp.float32)
    sel_p = nms(valid_p, mpp_ref)
    sel_t = nms(valid_t, mtt_ref)

    def choice(cs, ci):
        def acc(t, carry):
            bts, bti = carry
            d = offs_ref[0, t]
            d_pos = jnp.where(d < 0, d + N, d)
            g_selt = _roll_flat(sel_t, d_pos)
            gcs = _roll_flat(cs, d_pos)
            gci = _roll_flat(ci, d_pos)
            gts = _roll_flat(ts, d_pos)
            cand = (mpt_ref[t] * sel_p * g_selt) > 0.5
            before = (gcs > ps) | ((gcs == ps) & (gci < flat))
            tj = flat + d
            tj = jnp.where(tj >= N, tj - N, tj)
            tj = jnp.where(tj < 0, tj + N, tj)
            better = (gts > bts) | ((gts == bts) & (tj < bti))
            take = cand & jnp.logical_not(before) & better
            bts = jnp.where(take, gts, bts)
            bti = jnp.where(take, tj, bti)
            return bts, bti

        _, bti = lax.fori_loop(0, N_MATCH, acc,
                               (jnp.full((ROWS, COLS), -1.0, jnp.float32),
                                jnp.full((ROWS, COLS), SENT, jnp.int32)))
        return bti

    def claims(ct):
        def acc(t, carry):
            ncs, nci = carry
            d = offs_ref[0, t]
            rd = jnp.where(d > 0, N - d, -d)
            gct = _roll_flat(ct, rd)
            gps = _roll_flat(ps, rd)
            gpi = flat + rd
            gpi = jnp.where(gpi >= N, gpi - N, gpi)
            chose_me = gct == flat
            better = chose_me & ((gps > ncs) | ((gps == ncs) & (gpi < nci)))
            ncs = jnp.where(better, gps, ncs)
            nci = jnp.where(better, gpi, nci)
            return ncs, nci

        return lax.fori_loop(0, N_MATCH, acc,
                             (jnp.full((ROWS, COLS), -1.0, jnp.float32),
                              jnp.full((ROWS, COLS), BIG_I, jnp.int32)))

    def match_body(carry):
        cs, ci, ct_prev, _ = carry
        ct = choice(cs, ci)
        ncs, nci = claims(ct)
        changed = jnp.sum(jnp.abs(ct - ct_prev)) > 0
        return ncs, nci, ct, changed

    _, _, ct, _ = lax.while_loop(
        lambda c: c[3], match_body,
        (jnp.full((ROWS, COLS), -1.0, jnp.float32),
         jnp.full((ROWS, COLS), BIG_I, jnp.int32),
         jnp.full((ROWS, COLS), SENT, jnp.int32),
         jnp.bool_(True)))

    tp = jnp.sum((ct != SENT).astype(jnp.int32))
    nsp = jnp.sum(sel_p).astype(jnp.int32)
    nst = jnp.sum(sel_t).astype(jnp.int32)
    fp = nsp - tp
    fn = nst - tp

    lane = lax.broadcasted_iota(jnp.int32, (1, 1, COLS), 2)
    out_ref[...] = (jnp.where(lane == 0, tp, 0)
                    + jnp.where(lane == 1, fp, 0)
                    + jnp.where(lane == 2, fn, 0))


@functools.partial(jax.jit, static_argnames=("interpret",))
def _run(pred_c, targ_c, offs, interpret=False):
    return pl.pallas_call(
        _body,
        grid=(32,),
        in_specs=[
            pl.BlockSpec((1, 4, ROWS, COLS), lambda s: (s, 0, 0, 0)),
            pl.BlockSpec((1, 4, ROWS, COLS), lambda s: (s, 0, 0, 0)),
            pl.BlockSpec(memory_space=pltpu.SMEM),
        ],
        out_specs=pl.BlockSpec((1, 1, COLS), lambda s: (s, 0, 0)),
        out_shape=jax.ShapeDtypeStruct((32, 1, COLS), jnp.int32),
        scratch_shapes=[
            pltpu.VMEM((N_NMS, ROWS, COLS), jnp.float32),
            pltpu.VMEM((N_NMS, ROWS, COLS), jnp.float32),
            pltpu.VMEM((N_MATCH, ROWS, COLS), jnp.float32),
        ],
        compiler_params=pltpu.CompilerParams(
            dimension_semantics=("arbitrary",)),
        interpret=interpret,
    )(pred_c, targ_c, offs)


def _rearrange(a):
    t = a.reshape(16, 32, 32, 4, 2, 4)
    t = t.transpose(0, 4, 5, 3, 1, 2)
    t = t[:, :, jnp.array([2, 0, 1, 3])]
    return t.reshape(32, 4, ROWS, COLS)


def kernel(predictions, targets, interpret=False):
    pred_c = _rearrange(predictions)
    targ_c = _rearrange(targets)
    offs = jnp.asarray(MATCH_OFFS + [0] * (COLS - N_MATCH),
                       dtype=jnp.int32).reshape(1, COLS)
    out = _run(pred_c, targ_c, offs, interpret=interpret)
    return out[:, 0, :3].reshape(16, 2, 3)


# speedup vs baseline: 43.6552x; 43.6552x over previous
import functools

import jax
import jax.numpy as jnp
from jax import lax
from jax.experimental import pallas as pl
from jax.experimental.pallas import tpu as pltpu

N = 4096
ROWS, COLS = 32, 128
THRESHOLD = 0.5
D_O = 0.74 * 1.4
D_H = 0.528 * 1.4
EXPAND = (3.0 / 4.0, 25.0 / 32.0, 25.0 / 32.0)
BIG_I = 1 << 24
SENT = 1 << 24


def _feasible_offsets():
    offs = []
    for dz in range(-2, 3):
        for dx in range(-2, 3):
            for dy in range(-2, 3):
                if dz == dx == dy == 0:
                    continue
                mind2 = ((max(abs(dz) - 1, 0) * EXPAND[0]) ** 2
                         + (max(abs(dx) - 1, 0) * EXPAND[1]) ** 2
                         + (max(abs(dy) - 1, 0) * EXPAND[2]) ** 2)
                if mind2 < D_O * D_O:
                    offs.append(dz * 1024 + dx * 32 + dy)
    return offs


NBR_OFFS = _feasible_offsets()
N_NMS = len(NBR_OFFS)
MATCH_OFFS = [0] + NBR_OFFS
N_MATCH = len(MATCH_OFFS)


def _roll_flat(a, d_pos):
    q = d_pos // COLS
    rem = d_pos % COLS
    b1 = pltpu.roll(a, (ROWS - q % ROWS) % ROWS, axis=0)
    b2 = pltpu.roll(b1, ROWS - 1, axis=0)
    c1 = pltpu.roll(b1, (COLS - rem) % COLS, axis=1)
    c2 = pltpu.roll(b2, (COLS - rem) % COLS, axis=1)
    col = lax.broadcasted_iota(jnp.int32, a.shape, 1)
    return jnp.where(col < COLS - rem, c1, c2)


def _body(pred_ref, targ_ref, offs_ref, out_ref, mpp_ref, mtt_ref, mpt_ref):
    s = pl.program_id(0)
    e = s % 2
    thr = jnp.where(e == 0, jnp.float32(D_O), jnp.float32(D_H))

    flat = (lax.broadcasted_iota(jnp.int32, (ROWS, COLS), 0) * COLS
            + lax.broadcasted_iota(jnp.int32, (ROWS, COLS), 1))
    zc = (flat // 1024).astype(jnp.float32)
    xc = ((flat // 32) % 32).astype(jnp.float32)
    yc = (flat % 32).astype(jnp.float32)

    def coords(ref):
        z = (ref[0, 0] + zc) * jnp.float32(EXPAND[0])
        x = (ref[0, 1] + xc) * jnp.float32(EXPAND[1])
        y = (ref[0, 2] + yc) * jnp.float32(EXPAND[2])
        return z, x, y, ref[0, 3]

    pz, px, py, ps = coords(pred_ref)
    tz, tx, ty, ts = coords(targ_ref)

    def close(az, ax, ay, bz, bx, by, d_pos):
        gz = _roll_flat(bz, d_pos)
        gx = _roll_flat(bx, d_pos)
        gy = _roll_flat(by, d_pos)
        dz = gz - az
        dx = gx - ax
        dy = gy - ay
        d2 = dz * dz + dx * dx + dy * dy
        return jnp.sqrt(d2) < thr

    def precompute(t, _):
        d = offs_ref[0, t]
        d_pos = jnp.where(d < 0, d + N, d)
        neg = d < 0
        cpp = close(pz, px, py, pz, px, py, d_pos)
        ctt = close(tz, tx, ty, tz, tx, ty, d_pos)
        cpt = close(pz, px, py, tz, tx, ty, d_pos)
        gps = _roll_flat(ps, d_pos)
        gts = _roll_flat(ts, d_pos)
        ep = (gps > ps) | ((gps == ps) & neg)
        et = (gts > ts) | ((gts == ts) & neg)
        mpt_ref[t] = cpt.astype(jnp.float32)

        @pl.when(t > 0)
        def _():
            mpp_ref[t - 1] = (cpp & ep).astype(jnp.float32)
            mtt_ref[t - 1] = (ctt & et).astype(jnp.float32)

        return 0

    lax.fori_loop(0, N_MATCH, precompute, 0)

    def nms(valid, mask_ref):
        def body(carry):
            sel, _ = carry

            def acc(t, sup):
                d = offs_ref[0, t + 1]
                d_pos = jnp.where(d < 0, d + N, d)
                return jnp.maximum(sup, mask_ref[t] * _roll_flat(sel, d_pos))

            sup = lax.fori_loop(0, N_NMS, acc, jnp.zeros_like(sel))
            new = valid * (1.0 - sup)
            changed = jnp.sum(jnp.abs(new - sel)) > 0.0
            return new, changed

        sel, _ = lax.while_loop(lambda c: c[1], body,
                                (valid, jnp.bool_(True)))
        return sel

    valid_p = (ps > THRESHOLD).astype(jnp.float32)
    valid_t = (ts > THRESHOLD).astype(jnp.float32)
    sel_p = nms(valid_p, mpp_ref)
    sel_t = nms(valid_t, mtt_ref)

    def choice(cs, ci):
        def acc(t, carry):
            bts, bti = carry
            d = offs_ref[0, t]
            d_pos = jnp.where(d < 0, d + N, d)
            g_selt = _roll_flat(sel_t, d_pos)
            gcs = _roll_flat(cs, d_pos)
            gci = _roll_flat(ci, d_pos)
            gts = _roll_flat(ts, d_pos)
            cand = (mpt_ref[t] * sel_p * g_selt) > 0.5
            before = (gcs > ps) | ((gcs == ps) & (gci < flat))
            tj = flat + d
            tj = jnp.where(tj >= N, tj - N, tj)
            tj = jnp.where(tj < 0, tj + N, tj)
            better = (gts > bts) | ((gts == bts) & (tj < bti))
            take = cand & jnp.logical_not(before) & better
            bts = jnp.where(take, gts, bts)
            bti = jnp.where(take, tj, bti)
            return bts, bti

        _, bti = lax.fori_loop(0, N_MATCH, acc,
                               (jnp.full((ROWS, COLS), -1.0, jnp.float32),
                                jnp.full((ROWS, COLS), SENT, jnp.int32)))
        return bti

    def claims(ct):
        def acc(t, carry):
            ncs, nci = carry
            d = offs_ref[0, t]
            rd = jnp.where(d > 0, N - d, -d)
            gct = _roll_flat(ct, rd)
            gps = _roll_flat(ps, rd)
            gpi = flat + rd
            gpi = jnp.where(gpi >= N, gpi - N, gpi)
            chose_me = gct == flat
            better = chose_me & ((gps > ncs) | ((gps == ncs) & (gpi < nci)))
            ncs = jnp.where(better, gps, ncs)
            nci = jnp.where(better, gpi, nci)
            return ncs, nci

        return lax.fori_loop(0, N_MATCH, acc,
                             (jnp.full((ROWS, COLS), -1.0, jnp.float32),
                              jnp.full((ROWS, COLS), BIG_I, jnp.int32)))

    def match_body(carry):
        cs, ci, ct_prev, _ = carry
        ct = choice(cs, ci)
        ncs, nci = claims(ct)
        changed = jnp.sum((ct != ct_prev).astype(jnp.int32)) > 0
        return ncs, nci, ct, changed

    _, _, ct, _ = lax.while_loop(
        lambda c: c[3], match_body,
        (jnp.full((ROWS, COLS), -1.0, jnp.float32),
         jnp.full((ROWS, COLS), BIG_I, jnp.int32),
         jnp.full((ROWS, COLS), SENT, jnp.int32),
         jnp.bool_(True)))

    tp = jnp.sum((ct != SENT).astype(jnp.int32))
    nsp = jnp.sum(sel_p).astype(jnp.int32)
    nst = jnp.sum(sel_t).astype(jnp.int32)
    fp = nsp - tp
    fn = nst - tp

    lane = lax.broadcasted_iota(jnp.int32, (1, 1, COLS), 2)
    out_ref[...] = (jnp.where(lane == 0, tp, 0)
                    + jnp.where(lane == 1, fp, 0)
                    + jnp.where(lane == 2, fn, 0))


@functools.partial(jax.jit, static_argnames=("interpret",))
def _run(pred_c, targ_c, offs, interpret=False):
    return pl.pallas_call(
        _body,
        grid=(32,),
        in_specs=[
            pl.BlockSpec((1, 4, ROWS, COLS), lambda s: (s, 0, 0, 0)),
            pl.BlockSpec((1, 4, ROWS, COLS), lambda s: (s, 0, 0, 0)),
            pl.BlockSpec(memory_space=pltpu.SMEM),
        ],
        out_specs=pl.BlockSpec((1, 1, COLS), lambda s: (s, 0, 0)),
        out_shape=jax.ShapeDtypeStruct((32, 1, COLS), jnp.int32),
        scratch_shapes=[
            pltpu.VMEM((N_NMS, ROWS, COLS), jnp.float32),
            pltpu.VMEM((N_NMS, ROWS, COLS), jnp.float32),
            pltpu.VMEM((N_MATCH, ROWS, COLS), jnp.float32),
        ],
        compiler_params=pltpu.CompilerParams(
            dimension_semantics=("arbitrary",)),
        interpret=interpret,
    )(pred_c, targ_c, offs)


def _rearrange(a):
    t = a.reshape(16, 32, 32, 4, 2, 4)
    t = t.transpose(0, 4, 5, 3, 1, 2)
    t = t[:, :, jnp.array([2, 0, 1, 3])]
    return t.reshape(32, 4, ROWS, COLS)


def kernel(predictions, targets, interpret=False):
    pred_c = _rearrange(predictions)
    targ_c = _rearrange(targets)
    offs = jnp.asarray(MATCH_OFFS + [0] * (COLS - N_MATCH),
                       dtype=jnp.int32).reshape(1, COLS)
    out = _run(pred_c, targ_c, offs, interpret=interpret)
    return out[:, 0, :3].reshape(16, 2, 3)


# speedup vs baseline: 538.6138x; 12.3379x over previous
import functools

import jax
import jax.numpy as jnp
from jax import lax
from jax.experimental import pallas as pl
from jax.experimental.pallas import tpu as pltpu

N = 4096
ROWS, COLS = 32, 128
THRESHOLD = 0.5
D_O = 0.74 * 1.4
D_H = 0.528 * 1.4
EXPAND = (3.0 / 4.0, 25.0 / 32.0, 25.0 / 32.0)
SENT = 1 << 24
BIG_I = 1 << 24


def _feasible_offsets(dist):
    offs = []
    for dz in range(-2, 3):
        for dx in range(-2, 3):
            for dy in range(-2, 3):
                if dz == dx == dy == 0:
                    continue
                mind2 = ((max(abs(dz) - 1, 0) * EXPAND[0]) ** 2
                         + (max(abs(dx) - 1, 0) * EXPAND[1]) ** 2
                         + (max(abs(dy) - 1, 0) * EXPAND[2]) ** 2)
                if mind2 < dist * dist:
                    offs.append(dz * 1024 + dx * 32 + dy)
    return offs


OFFS_O = _feasible_offsets(D_O)
OFFS_H = _feasible_offsets(D_H)


def _roll_flat(a, d):
    d = d % N
    q, rem = divmod(d, COLS)
    b1 = a if q % ROWS == 0 else pltpu.roll(a, (ROWS - q % ROWS) % ROWS, axis=0)
    if rem == 0:
        return b1
    b2 = pltpu.roll(b1, ROWS - 1, axis=0)
    c1 = pltpu.roll(b1, COLS - rem, axis=1)
    c2 = pltpu.roll(b2, COLS - rem, axis=1)
    col = lax.broadcasted_iota(jnp.int32, a.shape, 1)
    return jnp.where(col < COLS - rem, c1, c2)


def _make_body(thr, offs):
    n_nms = len(offs)
    moffs = [0] + offs
    n_match = len(moffs)
    thr = float(thr)

    def body(pred_ref, targ_ref, out_ref,
             mpp_ref, mtt_ref, cm_ref, rts_ref, rps_ref):
        flat = (lax.broadcasted_iota(jnp.int32, (ROWS, COLS), 0) * COLS
                + lax.broadcasted_iota(jnp.int32, (ROWS, COLS), 1))
        zc = (flat // 1024).astype(jnp.float32)
        xc = ((flat // 32) % 32).astype(jnp.float32)
        yc = (flat % 32).astype(jnp.float32)

        def coords(ref):
            z = (ref[0, 0] + zc) * jnp.float32(EXPAND[0])
            x = (ref[0, 1] + xc) * jnp.float32(EXPAND[1])
            y = (ref[0, 2] + yc) * jnp.float32(EXPAND[2])
            return z, x, y, ref[0, 3]

        pz, px, py, ps = coords(pred_ref)
        tz, tx, ty, ts = coords(targ_ref)

        def close(az, ax, ay, bz, bx, by, d):
            gz = _roll_flat(bz, d)
            gx = _roll_flat(bx, d)
            gy = _roll_flat(by, d)
            dz = gz - az
            dx = gx - ax
            dy = gy - ay
            return jnp.sqrt(dz * dz + dx * dx + dy * dy) < jnp.float32(thr)

        for t, d in enumerate(moffs):
            cpt = close(pz, px, py, tz, tx, ty, d)
            cm_ref[t] = cpt.astype(jnp.float32)
            rts_ref[t] = _roll_flat(ts, d)
            rps_ref[t] = _roll_flat(ps, -d)
            if t > 0:
                cpp = close(pz, px, py, pz, px, py, d)
                ctt = close(tz, tx, ty, tz, tx, ty, d)
                gps = _roll_flat(ps, d)
                gts = rts_ref[t]
                if d < 0:
                    ep = (gps > ps) | (gps == ps)
                    et = (gts > ts) | (gts == ts)
                else:
                    ep = gps > ps
                    et = gts > ts
                mpp_ref[t - 1] = (cpp & ep).astype(jnp.float32)
                mtt_ref[t - 1] = (ctt & et).astype(jnp.float32)

        def nms(valid, mask_ref):
            def bodyw(carry):
                sel, _ = carry
                sup = jnp.zeros_like(sel)
                for t, d in enumerate(offs):
                    sup = jnp.maximum(sup, mask_ref[t] * _roll_flat(sel, d))
                new = valid * (1.0 - sup)
                changed = jnp.sum((new != sel).astype(jnp.int32)) > 0
                return new, changed

            sel, _ = lax.while_loop(lambda c: c[1], bodyw,
                                    (valid, jnp.bool_(True)))
            return sel

        sel_p = nms((ps > THRESHOLD).astype(jnp.float32), mpp_ref)
        sel_t = nms((ts > THRESHOLD).astype(jnp.float32), mtt_ref)

        for t, d in enumerate(moffs):
            cm_ref[t] = cm_ref[t] * sel_p * _roll_flat(sel_t, d)

        def choice(cs, ci):
            bts = jnp.full((ROWS, COLS), -1.0, jnp.float32)
            bti = jnp.full((ROWS, COLS), SENT, jnp.int32)
            for t, d in enumerate(moffs):
                gcs = _roll_flat(cs, d)
                gci = _roll_flat(ci, d)
                gts = rts_ref[t]
                cand = cm_ref[t] > 0.5
                before = (gcs > ps) | ((gcs == ps) & (gci < flat))
                tj = (flat + d) % N if d else flat
                better = (gts > bts) | ((gts == bts) & (tj < bti))
                take = cand & jnp.logical_not(before) & better
                bts = jnp.where(take, gts, bts)
                bti = jnp.where(take, tj, bti)
            return bti

        def claims(ct):
            ncs = jnp.full((ROWS, COLS), -1.0, jnp.float32)
            nci = jnp.full((ROWS, COLS), BIG_I, jnp.int32)
            for t, d in enumerate(moffs):
                gct = _roll_flat(ct, -d)
                gps = rps_ref[t]
                gpi = (flat - d) % N if d else flat
                chose_me = gct == flat
                better = chose_me & ((gps > ncs) | ((gps == ncs) & (gpi < nci)))
                ncs = jnp.where(better, gps, ncs)
                nci = jnp.where(better, gpi, nci)
            return ncs, nci

        def match_body(carry):
            cs, ci, ct_prev, _ = carry
            ct = choice(cs, ci)
            ncs, nci = claims(ct)
            changed = jnp.sum((ct != ct_prev).astype(jnp.int32)) > 0
            return ncs, nci, ct, changed

        _, _, ct, _ = lax.while_loop(
            lambda c: c[3], match_body,
            (jnp.full((ROWS, COLS), -1.0, jnp.float32),
             jnp.full((ROWS, COLS), BIG_I, jnp.int32),
             jnp.full((ROWS, COLS), SENT, jnp.int32),
             jnp.bool_(True)))

        tp = jnp.sum((ct != SENT).astype(jnp.int32))
        nsp = jnp.sum(sel_p).astype(jnp.int32)
        nst = jnp.sum(sel_t).astype(jnp.int32)
        fp = nsp - tp
        fn = nst - tp

        lane = lax.broadcasted_iota(jnp.int32, (1, 1, COLS), 2)
        out_ref[...] = (jnp.where(lane == 0, tp, 0)
                        + jnp.where(lane == 1, fp, 0)
                        + jnp.where(lane == 2, fn, 0))

    return body, n_nms, n_match


def _make_call(thr, offs):
    body, n_nms, n_match = _make_body(thr, offs)
    return pl.pallas_call(
        body,
        grid=(16,),
        in_specs=[
            pl.BlockSpec((1, 4, ROWS, COLS), lambda s: (s, 0, 0, 0)),
            pl.BlockSpec((1, 4, ROWS, COLS), lambda s: (s, 0, 0, 0)),
        ],
        out_specs=pl.BlockSpec((1, 1, COLS), lambda s: (s, 0, 0)),
        out_shape=jax.ShapeDtypeStruct((16, 1, COLS), jnp.int32),
        scratch_shapes=[
            pltpu.VMEM((n_nms, ROWS, COLS), jnp.float32),
            pltpu.VMEM((n_nms, ROWS, COLS), jnp.float32),
            pltpu.VMEM((n_match, ROWS, COLS), jnp.float32),
            pltpu.VMEM((n_match, ROWS, COLS), jnp.float32),
            pltpu.VMEM((n_match, ROWS, COLS), jnp.float32),
        ],
        compiler_params=pltpu.CompilerParams(
            dimension_semantics=("arbitrary",)),
    )


@jax.jit
def _run(pred_c, targ_c):
    out_o = _make_call(D_O, OFFS_O)(pred_c[0::2], targ_c[0::2])
    out_h = _make_call(D_H, OFFS_H)(pred_c[1::2], targ_c[1::2])
    return jnp.stack([out_o[:, 0, :3], out_h[:, 0, :3]], axis=1)


def _rearrange(a):
    t = a.reshape(16, 32, 32, 4, 2, 4)
    t = t.transpose(0, 4, 5, 3, 1, 2)
    t = t[:, :, jnp.array([2, 0, 1, 3])]
    return t.reshape(32, 4, ROWS, COLS)


def kernel(predictions, targets):
    pred_c = _rearrange(predictions)
    targ_c = _rearrange(targets)
    return _run(pred_c, targ_c)


# speedup vs baseline: 564.3242x; 1.0477x over previous
import functools

import numpy as np
import jax
import jax.numpy as jnp
from jax import lax
from jax.experimental import pallas as pl
from jax.experimental.pallas import tpu as pltpu

N = 4096
ROWS, COLS = 32, 128
THRESHOLD = 0.5
D_O = 0.74 * 1.4
D_H = 0.528 * 1.4
EXPAND = (3.0 / 4.0, 25.0 / 32.0, 25.0 / 32.0)
SENT = 1 << 24
BIG_I = 1 << 24


def _feasible_offsets(dist):
    offs = []
    for dz in range(-2, 3):
        for dx in range(-2, 3):
            for dy in range(-2, 3):
                if dz == dx == dy == 0:
                    continue
                mind2 = ((max(abs(dz) - 1, 0) * EXPAND[0]) ** 2
                         + (max(abs(dx) - 1, 0) * EXPAND[1]) ** 2
                         + (max(abs(dy) - 1, 0) * EXPAND[2]) ** 2)
                if mind2 < dist * dist:
                    offs.append(dz * 1024 + dx * 32 + dy)
    return offs


OFFS_O = _feasible_offsets(D_O)
OFFS_H = _feasible_offsets(D_H)


def _sqrt_lt_bound(thr):
    thr = np.float32(thr)

    def fromb(b):
        return np.frombuffer(np.uint32(b).tobytes(), np.float32)[0]

    b = int(np.frombuffer((thr * thr).tobytes(), np.uint32)[0])
    while np.sqrt(fromb(b)) >= thr:
        b -= 1
    while np.sqrt(fromb(b)) < thr:
        b += 1
    return float(fromb(b))


def _roll_flat(a, d):
    d = d % N
    q, rem = divmod(d, COLS)
    b1 = a if q % ROWS == 0 else pltpu.roll(a, (ROWS - q % ROWS) % ROWS, axis=0)
    if rem == 0:
        return b1
    b2 = pltpu.roll(b1, ROWS - 1, axis=0)
    c1 = pltpu.roll(b1, COLS - rem, axis=1)
    c2 = pltpu.roll(b2, COLS - rem, axis=1)
    col = lax.broadcasted_iota(jnp.int32, a.shape, 1)
    return jnp.where(col < COLS - rem, c1, c2)


def _make_body(thr, offs):
    n_nms = len(offs)
    moffs = [0] + offs
    n_match = len(moffs)
    d2_bound = _sqrt_lt_bound(thr)

    def body(pred_ref, targ_ref, out_ref,
             mpp_ref, mtt_ref, cm_ref, rts_ref, rps_ref):
        flat = (lax.broadcasted_iota(jnp.int32, (ROWS, COLS), 0) * COLS
                + lax.broadcasted_iota(jnp.int32, (ROWS, COLS), 1))
        zc = (flat // 1024).astype(jnp.float32)
        xc = ((flat // 32) % 32).astype(jnp.float32)
        yc = (flat % 32).astype(jnp.float32)

        def coords(ref):
            z = (ref[0, 0] + zc) * jnp.float32(EXPAND[0])
            x = (ref[0, 1] + xc) * jnp.float32(EXPAND[1])
            y = (ref[0, 2] + yc) * jnp.float32(EXPAND[2])
            return z, x, y, ref[0, 3]

        pz, px, py, ps = coords(pred_ref)
        tz, tx, ty, ts = coords(targ_ref)

        def close(az, ax, ay, gz, gx, gy):
            dz = gz - az
            dx = gx - ax
            dy = gy - ay
            return (dz * dz + dx * dx + dy * dy) < jnp.float32(d2_bound)

        for t, d in enumerate(moffs):
            gtz = _roll_flat(tz, d)
            gtx = _roll_flat(tx, d)
            gty = _roll_flat(ty, d)
            gts = _roll_flat(ts, d)
            cm_ref[t] = close(pz, px, py, gtz, gtx, gty).astype(jnp.float32)
            rts_ref[t] = gts
            rps_ref[t] = _roll_flat(ps, -d)
            if t > 0:
                gpz = _roll_flat(pz, d)
                gpx = _roll_flat(px, d)
                gpy = _roll_flat(py, d)
                gps = _roll_flat(ps, d)
                cpp = close(pz, px, py, gpz, gpx, gpy)
                ctt = close(tz, tx, ty, gtz, gtx, gty)
                if d < 0:
                    ep = gps >= ps
                    et = gts >= ts
                else:
                    ep = gps > ps
                    et = gts > ts
                mpp_ref[t - 1] = (cpp & ep).astype(jnp.float32)
                mtt_ref[t - 1] = (ctt & et).astype(jnp.float32)

        def nms(valid, mask_ref):
            def bodyw(carry):
                sel, _ = carry
                sup = jnp.zeros_like(sel)
                for t, d in enumerate(offs):
                    sup = jnp.maximum(sup, mask_ref[t] * _roll_flat(sel, d))
                new = valid * (1.0 - sup)
                changed = jnp.sum((new != sel).astype(jnp.int32)) > 0
                return new, changed

            sel, _ = lax.while_loop(lambda c: c[1], bodyw,
                                    (valid, jnp.bool_(True)))
            return sel

        sel_p = nms((ps > THRESHOLD).astype(jnp.float32), mpp_ref)
        sel_t = nms((ts > THRESHOLD).astype(jnp.float32), mtt_ref)

        for t, d in enumerate(moffs):
            cand = (cm_ref[t] * sel_p * _roll_flat(sel_t, d)) > 0.5
            rts_ref[t] = jnp.where(cand, rts_ref[t], -1.0)

        def choice(cs, ci):
            bts = jnp.full((ROWS, COLS), -1.0, jnp.float32)
            bti = jnp.full((ROWS, COLS), SENT, jnp.int32)
            for t, d in enumerate(moffs):
                gcs = _roll_flat(cs, d)
                gci = _roll_flat(ci, d)
                gts = rts_ref[t]
                before = (gcs > ps) | ((gcs == ps) & (gci < flat))
                tj = (flat + d) & (N - 1) if d else flat
                better = (gts > bts) | ((gts == bts) & (tj < bti))
                take = (gts >= 0) & jnp.logical_not(before) & better
                bts = jnp.where(take, gts, bts)
                bti = jnp.where(take, tj, bti)
            return bti

        def claims(ct):
            ncs = jnp.full((ROWS, COLS), -1.0, jnp.float32)
            nci = jnp.full((ROWS, COLS), BIG_I, jnp.int32)
            for t, d in enumerate(moffs):
                gct = _roll_flat(ct, -d)
                gps = rps_ref[t]
                gpi = (flat - d) & (N - 1) if d else flat
                chose_me = gct == flat
                better = chose_me & ((gps > ncs) | ((gps == ncs) & (gpi < nci)))
                ncs = jnp.where(better, gps, ncs)
                nci = jnp.where(better, gpi, nci)
            return ncs, nci

        def match_body(carry):
            cs, ci, ct_prev, _ = carry
            ct = choice(cs, ci)
            ncs, nci = claims(ct)
            changed = jnp.sum((ct != ct_prev).astype(jnp.int32)) > 0
            return ncs, nci, ct, changed

        _, _, ct, _ = lax.while_loop(
            lambda c: c[3], match_body,
            (jnp.full((ROWS, COLS), -1.0, jnp.float32),
             jnp.full((ROWS, COLS), BIG_I, jnp.int32),
             jnp.full((ROWS, COLS), SENT, jnp.int32),
             jnp.bool_(True)))

        tp = jnp.sum((ct != SENT).astype(jnp.int32))
        nsp = jnp.sum(sel_p).astype(jnp.int32)
        nst = jnp.sum(sel_t).astype(jnp.int32)
        fp = nsp - tp
        fn = nst - tp

        lane = lax.broadcasted_iota(jnp.int32, (1, 1, COLS), 2)
        out_ref[...] = (jnp.where(lane == 0, tp, 0)
                        + jnp.where(lane == 1, fp, 0)
                        + jnp.where(lane == 2, fn, 0))

    return body, n_nms, n_match


def _make_call(thr, offs):
    body, n_nms, n_match = _make_body(thr, offs)
    return pl.pallas_call(
        body,
        grid=(16,),
        in_specs=[
            pl.BlockSpec((1, 4, ROWS, COLS), lambda s: (s, 0, 0, 0)),
            pl.BlockSpec((1, 4, ROWS, COLS), lambda s: (s, 0, 0, 0)),
        ],
        out_specs=pl.BlockSpec((1, 1, COLS), lambda s: (s, 0, 0)),
        out_shape=jax.ShapeDtypeStruct((16, 1, COLS), jnp.int32),
        scratch_shapes=[
            pltpu.VMEM((n_nms, ROWS, COLS), jnp.float32),
            pltpu.VMEM((n_nms, ROWS, COLS), jnp.float32),
            pltpu.VMEM((n_match, ROWS, COLS), jnp.float32),
            pltpu.VMEM((n_match, ROWS, COLS), jnp.float32),
            pltpu.VMEM((n_match, ROWS, COLS), jnp.float32),
        ],
        compiler_params=pltpu.CompilerParams(
            dimension_semantics=("arbitrary",)),
    )


@jax.jit
def _run(pred_c, targ_c):
    out_o = _make_call(D_O, OFFS_O)(pred_c[0::2], targ_c[0::2])
    out_h = _make_call(D_H, OFFS_H)(pred_c[1::2], targ_c[1::2])
    return jnp.stack([out_o[:, 0, :3], out_h[:, 0, :3]], axis=1)


def _rearrange(a):
    t = a.reshape(16, 32, 32, 4, 2, 4)
    t = t.transpose(0, 4, 5, 3, 1, 2)
    t = t[:, :, jnp.array([2, 0, 1, 3])]
    return t.reshape(32, 4, ROWS, COLS)


def kernel(predictions, targets):
    pred_c = _rearrange(predictions)
    targ_c = _rearrange(targets)
    return _run(pred_c, targ_c)


# speedup vs baseline: 564.4440x; 1.0002x over previous
import functools

import jax
import jax.numpy as jnp
from jax import lax
from jax.experimental import pallas as pl
from jax.experimental.pallas import tpu as pltpu

N = 4096
ROWS, COLS = 32, 128
THRESHOLD = 0.5
D_O = 0.74 * 1.4
D_H = 0.528 * 1.4
EXPAND = (3.0 / 4.0, 25.0 / 32.0, 25.0 / 32.0)
SENT = 1 << 24
BIG_I = 1 << 24


def _feasible_offsets(dist):
    offs = []
    for dz in range(-2, 3):
        for dx in range(-2, 3):
            for dy in range(-2, 3):
                if dz == dx == dy == 0:
                    continue
                mind2 = ((max(abs(dz) - 1, 0) * EXPAND[0]) ** 2
                         + (max(abs(dx) - 1, 0) * EXPAND[1]) ** 2
                         + (max(abs(dy) - 1, 0) * EXPAND[2]) ** 2)
                if mind2 < dist * dist:
                    offs.append(dz * 1024 + dx * 32 + dy)
    return offs


OFFS_O = _feasible_offsets(D_O)
OFFS_H = _feasible_offsets(D_H)


def _roll_flat(a, d):
    d = d % N
    q, rem = divmod(d, COLS)
    b1 = a if q % ROWS == 0 else pltpu.roll(a, (ROWS - q % ROWS) % ROWS, axis=0)
    if rem == 0:
        return b1
    b2 = pltpu.roll(b1, ROWS - 1, axis=0)
    c1 = pltpu.roll(b1, COLS - rem, axis=1)
    c2 = pltpu.roll(b2, COLS - rem, axis=1)
    col = lax.broadcasted_iota(jnp.int32, a.shape, 1)
    return jnp.where(col < COLS - rem, c1, c2)


def _make_body(thr, offs):
    n_nms = len(offs)
    moffs = [0] + offs
    n_match = len(moffs)
    thr = float(thr)

    def body(pred_ref, targ_ref, out_ref,
             mpp_ref, mtt_ref, cm_ref, rts_ref, rps_ref):
        flat = (lax.broadcasted_iota(jnp.int32, (ROWS, COLS), 0) * COLS
                + lax.broadcasted_iota(jnp.int32, (ROWS, COLS), 1))
        zc = (flat // 1024).astype(jnp.float32)
        xc = ((flat // 32) % 32).astype(jnp.float32)
        yc = (flat % 32).astype(jnp.float32)

        def coords(ref):
            z = (ref[0, 0] + zc) * jnp.float32(EXPAND[0])
            x = (ref[0, 1] + xc) * jnp.float32(EXPAND[1])
            y = (ref[0, 2] + yc) * jnp.float32(EXPAND[2])
            return z, x, y, ref[0, 3]

        pz, px, py, ps = coords(pred_ref)
        tz, tx, ty, ts = coords(targ_ref)

        def close(az, ax, ay, gz, gx, gy):
            dz = gz - az
            dx = gx - ax
            dy = gy - ay
            return jnp.sqrt(dz * dz + dx * dx + dy * dy) < jnp.float32(thr)

        for t, d in enumerate(moffs):
            gtz = _roll_flat(tz, d)
            gtx = _roll_flat(tx, d)
            gty = _roll_flat(ty, d)
            gts = _roll_flat(ts, d)
            cm_ref[t] = close(pz, px, py, gtz, gtx, gty).astype(jnp.float32)
            rts_ref[t] = gts
            rps_ref[t] = _roll_flat(ps, -d)
            if t > 0:
                gpz = _roll_flat(pz, d)
                gpx = _roll_flat(px, d)
                gpy = _roll_flat(py, d)
                gps = _roll_flat(ps, d)
                cpp = close(pz, px, py, gpz, gpx, gpy)
                ctt = close(tz, tx, ty, gtz, gtx, gty)
                if d < 0:
                    ep = gps >= ps
                    et = gts >= ts
                else:
                    ep = gps > ps
                    et = gts > ts
                mpp_ref[t - 1] = (cpp & ep).astype(jnp.float32)
                mtt_ref[t - 1] = (ctt & et).astype(jnp.float32)

        def nms(valid, mask_ref):
            def bodyw(carry):
                sel, _ = carry
                sup = jnp.zeros_like(sel)
                for t, d in enumerate(offs):
                    sup = jnp.maximum(sup, mask_ref[t] * _roll_flat(sel, d))
                new = valid * (1.0 - sup)
                changed = jnp.sum((new != sel).astype(jnp.int32)) > 0
                return new, changed

            sel, _ = lax.while_loop(lambda c: c[1], bodyw,
                                    (valid, jnp.bool_(True)))
            return sel

        sel_p = nms((ps > THRESHOLD).astype(jnp.float32), mpp_ref)
        sel_t = nms((ts > THRESHOLD).astype(jnp.float32), mtt_ref)

        for t, d in enumerate(moffs):
            cand = (cm_ref[t] * sel_p * _roll_flat(sel_t, d)) > 0.5
            rts_ref[t] = jnp.where(cand, rts_ref[t], -1.0)

        def choice(cs, ci):
            bts = jnp.full((ROWS, COLS), -1.0, jnp.float32)
            bti = jnp.full((ROWS, COLS), SENT, jnp.int32)
            for t, d in enumerate(moffs):
                gcs = _roll_flat(cs, d)
                gci = _roll_flat(ci, d)
                gts = rts_ref[t]
                before = (gcs > ps) | ((gcs == ps) & (gci < flat))
                tj = (flat + d) & (N - 1) if d else flat
                better = (gts > bts) | ((gts == bts) & (tj < bti))
                take = (gts >= 0) & jnp.logical_not(before) & better
                bts = jnp.where(take, gts, bts)
                bti = jnp.where(take, tj, bti)
            return bti

        def claims(ct):
            ncs = jnp.full((ROWS, COLS), -1.0, jnp.float32)
            nci = jnp.full((ROWS, COLS), BIG_I, jnp.int32)
            for t, d in enumerate(moffs):
                gct = _roll_flat(ct, -d)
                gps = rps_ref[t]
                gpi = (flat - d) & (N - 1) if d else flat
                chose_me = gct == flat
                better = chose_me & ((gps > ncs) | ((gps == ncs) & (gpi < nci)))
                ncs = jnp.where(better, gps, ncs)
                nci = jnp.where(better, gpi, nci)
            return ncs, nci

        def match_body(carry):
            cs, ci, ct_prev, _ = carry
            ct = choice(cs, ci)
            ncs, nci = claims(ct)
            changed = jnp.sum((ct != ct_prev).astype(jnp.int32)) > 0
            return ncs, nci, ct, changed

        _, _, ct, _ = lax.while_loop(
            lambda c: c[3], match_body,
            (jnp.full((ROWS, COLS), -1.0, jnp.float32),
             jnp.full((ROWS, COLS), BIG_I, jnp.int32),
             jnp.full((ROWS, COLS), SENT, jnp.int32),
             jnp.bool_(True)))

        tp = jnp.sum((ct != SENT).astype(jnp.int32))
        nsp = jnp.sum(sel_p).astype(jnp.int32)
        nst = jnp.sum(sel_t).astype(jnp.int32)
        fp = nsp - tp
        fn = nst - tp

        lane = lax.broadcasted_iota(jnp.int32, (1, 1, COLS), 2)
        out_ref[...] = (jnp.where(lane == 0, tp, 0)
                        + jnp.where(lane == 1, fp, 0)
                        + jnp.where(lane == 2, fn, 0))

    return body, n_nms, n_match


def _make_call(thr, offs):
    body, n_nms, n_match = _make_body(thr, offs)
    return pl.pallas_call(
        body,
        grid=(16,),
        in_specs=[
            pl.BlockSpec((1, 4, ROWS, COLS), lambda s: (s, 0, 0, 0)),
            pl.BlockSpec((1, 4, ROWS, COLS), lambda s: (s, 0, 0, 0)),
        ],
        out_specs=pl.BlockSpec((1, 1, COLS), lambda s: (s, 0, 0)),
        out_shape=jax.ShapeDtypeStruct((16, 1, COLS), jnp.int32),
        scratch_shapes=[
            pltpu.VMEM((n_nms, ROWS, COLS), jnp.float32),
            pltpu.VMEM((n_nms, ROWS, COLS), jnp.float32),
            pltpu.VMEM((n_match, ROWS, COLS), jnp.float32),
            pltpu.VMEM((n_match, ROWS, COLS), jnp.float32),
            pltpu.VMEM((n_match, ROWS, COLS), jnp.float32),
        ],
        compiler_params=pltpu.CompilerParams(
            dimension_semantics=("arbitrary",)),
    )


@jax.jit
def _run(pred_c, targ_c):
    out_o = _make_call(D_O, OFFS_O)(pred_c[0::2], targ_c[0::2])
    out_h = _make_call(D_H, OFFS_H)(pred_c[1::2], targ_c[1::2])
    return jnp.stack([out_o[:, 0, :3], out_h[:, 0, :3]], axis=1)


def _rearrange(a):
    t = a.reshape(16, 32, 32, 4, 2, 4)
    t = t.transpose(0, 4, 5, 3, 1, 2)
    t = t[:, :, jnp.array([2, 0, 1, 3])]
    return t.reshape(32, 4, ROWS, COLS)


def kernel(predictions, targets):
    pred_c = _rearrange(predictions)
    targ_c = _rearrange(targets)
    return _run(pred_c, targ_c)
